# Initial kernel scaffold; baseline (speedup 1.0000x reference)
import jax
import jax.numpy as jnp
from jax.experimental import pallas as pl


def kernel(vfeat, efeat, bias_in, bias_con, W_vtx1, b_vtx1, cs_emb, un_emb, W_kv, b_kv, W_vv, b_vv, W_qe, b_qe, W_ke, b_ke, W_ve, b_ve, W_qv, b_qv, ln1_g, ln1_b, ln2_g, ln2_b, W_l1, b_l1, W_l2, b_l2, W_l3, b_l3, W_l4, b_l4, W_mlp, b_mlp, centrality_values, uniqueness, in_src, in_dst, con_src, con_dst):
    raise NotImplementedError("write your pallas kernel here")


# speedup vs baseline: 7.4351x; 7.4351x over previous
import functools
import math

import jax
import jax.numpy as jnp
from jax.experimental import pallas as pl
from jax.experimental.pallas import tpu as pltpu


def _ln(x, g, b):
    m = jnp.mean(x, axis=-1, keepdims=True)
    v = jnp.mean((x - m) * (x - m), axis=-1, keepdims=True)
    return (x - m) * jax.lax.rsqrt(v + 1e-5) * g + b


def _vtx_body(K, vfeat_ref, cent_ref, uniq_ref, Wv_ref, bv_ref, cs_ref, un_ref,
              Wk_ref, bk_ref, Wvv_ref, bvv_ref, Wq2_ref, bq2_ref,
              feat_ref, k_ref, v_ref, q2_ref):
    x = vfeat_ref[...]
    R = x.shape[0]
    f = jnp.dot(x, Wv_ref[...], preferred_element_type=jnp.float32) + bv_ref[...]
    oh_c = (jax.lax.broadcasted_iota(jnp.int32, (R, K), 1) == cent_ref[...]).astype(jnp.float32)
    f = f + jnp.dot(oh_c, cs_ref[...], preferred_element_type=jnp.float32)
    oh_u = (jax.lax.broadcasted_iota(jnp.int32, (R, K), 1) == uniq_ref[...]).astype(jnp.float32)
    f = f + jnp.dot(oh_u, un_ref[...], preferred_element_type=jnp.float32)
    feat_ref[...] = f
    k_ref[...] = jnp.dot(f, Wk_ref[...], preferred_element_type=jnp.float32) + bk_ref[...]
    v_ref[...] = jnp.dot(f, Wvv_ref[...], preferred_element_type=jnp.float32) + bvv_ref[...]
    q2_ref[...] = jnp.dot(f, Wq2_ref[...], preferred_element_type=jnp.float32) + bq2_ref[...]


def _edge_body(inv_sqrt_qd,
               efeat_ref, k_ref, v_ref, C_ref,
               Wqe_ref, bqe_ref, Wl1_ref, bl1_ref, Wl2_ref, bl2_ref,
               ln1g_ref, ln1b_ref, Wke_ref, bke_ref, Wve_ref, bve_ref,
               feat_e_ref, k2_ref, v2_ref):
    ef = efeat_ref[...]
    q = jnp.dot(ef, Wqe_ref[...], preferred_element_type=jnp.float32) + bqe_ref[...]
    S = jax.lax.dot_general(q, k_ref[...], (((1,), (1,)), ((), ())),
                            preferred_element_type=jnp.float32)
    A = jnp.where(S >= 0, S, 0.01 * S) * inv_sqrt_qd
    m = jnp.max(A, axis=1, keepdims=True)
    P = jnp.exp(A - m) * C_ref[...]
    s = jnp.sum(P, axis=1, keepdims=True)
    h = jnp.dot(P, v_ref[...], preferred_element_type=jnp.float32) / jnp.maximum(s, 1e-30)
    x = _ln(h + ef, ln1g_ref[...], ln1b_ref[...])
    f = jnp.dot(jax.nn.relu(jnp.dot(x, Wl1_ref[...], preferred_element_type=jnp.float32) + bl1_ref[...]),
                Wl2_ref[...], preferred_element_type=jnp.float32) + bl2_ref[...]
    fe = _ln(f + x, ln1g_ref[...], ln1b_ref[...])
    feat_e_ref[...] = fe
    k2_ref[...] = jnp.dot(fe, Wke_ref[...], preferred_element_type=jnp.float32) + bke_ref[...]
    v2_ref[...] = jnp.dot(fe, Wve_ref[...], preferred_element_type=jnp.float32) + bve_ref[...]


def _node_body(inv_sqrt_qd,
               featv_ref, q2_ref, k2_ref, v2_ref, C_ref,
               Wl3_ref, bl3_ref, Wl4_ref, bl4_ref,
               ln2g_ref, ln2b_ref, Wmlp_ref, bmlp_ref,
               out_ref):
    S = jax.lax.dot_general(q2_ref[...], k2_ref[...], (((1,), (1,)), ((), ())),
                            preferred_element_type=jnp.float32)
    A = jnp.where(S >= 0, S, 0.01 * S) * inv_sqrt_qd
    m = jnp.max(A, axis=1, keepdims=True)
    P = jnp.exp(A - m) * C_ref[...]
    s = jnp.sum(P, axis=1, keepdims=True)
    h = jnp.dot(P, v2_ref[...], preferred_element_type=jnp.float32) / jnp.maximum(s, 1e-30)
    y = _ln(h + featv_ref[...], ln2g_ref[...], ln2b_ref[...])
    f2 = jnp.dot(jax.nn.relu(jnp.dot(y, Wl3_ref[...], preferred_element_type=jnp.float32) + bl3_ref[...]),
                 Wl4_ref[...], preferred_element_type=jnp.float32) + bl4_ref[...]
    fv2 = _ln(f2 + y, ln2g_ref[...], ln2b_ref[...])
    out_ref[...] = jnp.dot(fv2, Wmlp_ref[...], preferred_element_type=jnp.float32) + bmlp_ref[...]


def _full(shape):
    return pl.BlockSpec(shape, lambda i: (0,) * len(shape))


def kernel(vfeat, efeat, bias_in, bias_con, W_vtx1, b_vtx1, cs_emb, un_emb,
           W_kv, b_kv, W_vv, b_vv, W_qe, b_qe, W_ke, b_ke, W_ve, b_ve,
           W_qv, b_qv, ln1_g, ln1_b, ln2_g, ln2_b, W_l1, b_l1, W_l2, b_l2,
           W_l3, b_l3, W_l4, b_l4, W_mlp, b_mlp,
           centrality_values, uniqueness, in_src, in_dst, con_src, con_dst):
    N, D = vfeat.shape
    M = efeat.shape[0]
    K = cs_emb.shape[0]
    H = W_mlp.shape[1]
    qd = W_kv.shape[1]
    inv_sqrt_qd = 1.0 / math.sqrt(qd)

    r2 = lambda a: a.reshape(1, -1)

    flat1 = in_dst * N + in_src
    C1 = jnp.zeros((M * N,), jnp.float32).at[flat1].add(jnp.exp(bias_in)).reshape(M, N)
    flat2 = con_dst * M + con_src
    C2 = jnp.zeros((N * M,), jnp.float32).at[flat2].add(jnp.exp(bias_con)).reshape(N, M)

    NB = 1000 if N % 1000 == 0 else N
    grid_n = N // NB
    cent2 = centrality_values.reshape(N, 1)
    uniq2 = uniqueness.reshape(N, 1)
    row_spec = pl.BlockSpec((NB, D), lambda i: (i, 0))
    idx_spec = pl.BlockSpec((NB, 1), lambda i: (i, 0))
    feat_v, kv, vv, q2 = pl.pallas_call(
        functools.partial(_vtx_body, K),
        grid=(grid_n,),
        in_specs=[row_spec, idx_spec, idx_spec,
                  _full((D, D)), _full((1, D)), _full((K, D)), _full((K, D)),
                  _full((D, D)), _full((1, D)), _full((D, D)), _full((1, D)),
                  _full((D, D)), _full((1, D))],
        out_specs=[row_spec, row_spec, row_spec, row_spec],
        out_shape=[jax.ShapeDtypeStruct((N, D), jnp.float32)] * 4,
    )(vfeat, cent2, uniq2, W_vtx1, r2(b_vtx1), cs_emb, un_emb,
      W_kv, r2(b_kv), W_vv, r2(b_vv), W_qv, r2(b_qv))

    MB = 80 if M % 80 == 0 else M
    grid_m = M // MB
    mrow_spec = pl.BlockSpec((MB, D), lambda i: (i, 0))
    feat_e, k2, v2 = pl.pallas_call(
        functools.partial(_edge_body, inv_sqrt_qd),
        grid=(grid_m,),
        in_specs=[mrow_spec, _full((N, D)), _full((N, D)),
                  pl.BlockSpec((MB, N), lambda i: (i, 0)),
                  _full((D, D)), _full((1, D)), _full((D, D)), _full((1, D)),
                  _full((D, D)), _full((1, D)), _full((1, D)), _full((1, D)),
                  _full((D, D)), _full((1, D)), _full((D, D)), _full((1, D))],
        out_specs=[mrow_spec, mrow_spec, mrow_spec],
        out_shape=[jax.ShapeDtypeStruct((M, D), jnp.float32)] * 3,
    )(efeat, kv, vv, C1,
      W_qe, r2(b_qe), W_l1, r2(b_l1), W_l2, r2(b_l2), r2(ln1_g), r2(ln1_b),
      W_ke, r2(b_ke), W_ve, r2(b_ve))

    out = pl.pallas_call(
        functools.partial(_node_body, inv_sqrt_qd),
        grid=(grid_n,),
        in_specs=[row_spec, row_spec, _full((M, D)), _full((M, D)),
                  pl.BlockSpec((NB, M), lambda i: (i, 0)),
                  _full((D, D)), _full((1, D)), _full((D, D)), _full((1, D)),
                  _full((1, D)), _full((1, D)), _full((D, H)), _full((1, H))],
        out_specs=pl.BlockSpec((NB, H), lambda i: (i, 0)),
        out_shape=jax.ShapeDtypeStruct((N, H), jnp.float32),
    )(feat_v, q2, k2, v2, C2,
      W_l3, r2(b_l3), W_l4, r2(b_l4), r2(ln2_g), r2(ln2_b), W_mlp, r2(b_mlp))

    return out
